# baseline (speedup 1.0000x reference)
import jax
import jax.numpy as jnp
from jax.experimental import pallas as pl

G = 128
P = 64
HIDDEN = 128
NF = 128
NG = 50
NI = 6
CUTOFF = 10.0
EMB = 128

_BF = jnp.bfloat16
_F32 = jnp.float32


def _sp(x):
    return jnp.maximum(x, 0.0) + jnp.log1p(jnp.exp(-jnp.abs(x)))


def _split(a):
    bits = jax.lax.bitcast_convert_type(a, jnp.uint32)
    r = (bits + jnp.uint32(0x7FFF) + ((bits >> 16) & jnp.uint32(1))) \
        & jnp.uint32(0xFFFF0000)
    hi32 = jax.lax.bitcast_convert_type(r, _F32)
    return hi32.astype(_BF), (a - hi32).astype(_BF)


def _dot3(ah, al, bh, bl):
    return (jnp.dot(ah, bh, preferred_element_type=_F32)
            + jnp.dot(al, bh, preferred_element_type=_F32)
            + jnp.dot(ah, bl, preferred_element_type=_F32))


def _mol_kernel(z_ref, pos_ref, embh_ref, embl_ref, w1h_ref, w1l_ref, b1c_ref,
                w2h_ref, w2l_ref, b2_ref, l1h_ref, l1l_ref,
                l2h_ref, l2l_ref, l2b_ref, lwh_ref, lwl_ref, lb_ref, out_ref):
    p = pos_ref[0]
    zc = z_ref[0]

    tt = jax.lax.broadcasted_iota(jnp.int32, (P, 100), 1)
    oh = (zc == tt).astype(_BF)
    h = (jnp.dot(oh, embh_ref[...], preferred_element_type=_F32)
         + jnp.dot(oh, embl_ref[...], preferred_element_type=_F32))

    diff = p[:, None, :] - p[None, :, :]
    r2 = jnp.sum(diff * diff, axis=-1, keepdims=True)
    ewf = jnp.sqrt(r2 + 1e-12).reshape(P * P, 1)

    step = CUTOFF / (NG - 1)
    offs = jax.lax.broadcasted_iota(jnp.int32, (1, NG), 1).astype(_F32) * step
    gcoeff = -0.5 / (step * step)
    ea = jnp.exp(gcoeff * (ewf - offs) ** 2)

    cf = 0.5 * (jnp.cos(ewf * (jnp.pi / CUTOFF)) + 1.0)
    cf = cf * (ewf < CUTOFF).astype(_F32)
    ii = jax.lax.broadcasted_iota(jnp.int32, (P, P, 1), 0)
    jj = jax.lax.broadcasted_iota(jnp.int32, (P, P, 1), 1)
    scale = cf * (ii != jj).astype(_F32).reshape(P * P, 1)

    eah, eal = _split(ea)
    h1c = _sp(_dot3(eah, eal, w1h_ref[...], w1l_ref[...])
              + b1c_ref[...])
    h1h, h1l = _split(h1c)

    for t in range(NI):
        sl = slice(t * NF, (t + 1) * NF)
        w = _dot3(h1h[:, sl], h1l[:, sl], w2h_ref[t], w2l_ref[t]) + b2_ref[t]
        w = w * scale
        x = _dot3(*_split(h), l1h_ref[t], l1l_ref[t])
        w3 = w.reshape(P, P, NF)
        agg = jnp.sum(w3 * x[None, :, :], axis=1)
        x2 = _sp(_dot3(*_split(agg), l2h_ref[t], l2l_ref[t]) + l2b_ref[t])
        x2 = _dot3(*_split(x2), lwh_ref[t], lwl_ref[t]) + lb_ref[t]
        h = h + x2

    out_ref[0] = jnp.sum(h, axis=0, keepdims=True) * (1.0 / P)


def _pool_kernel(hm_ref, pwh_ref, pwl_ref, pb_ref, out_ref):
    out_ref[...] = _dot3(*_split(hm_ref[...]), pwh_ref[...], pwl_ref[...]) \
        + pb_ref[...]


def kernel(z, pos, batch, emb_table, mlp_w1, mlp_b1, mlp_w2, mlp_b2,
           lin1_w, lin2_w, lin2_b, lin_w, lin_b, pool_w, pool_b):
    del batch
    ln2 = 0.6931471805599453
    z3 = z.reshape(G, P, 1).astype(jnp.int32)
    pos3 = pos.reshape(G, P, 3)
    w1c = jnp.transpose(mlp_w1, (1, 0, 2)).reshape(NG, NI * NF)
    b1c = mlp_b1.reshape(1, NI * NF)
    b2 = (mlp_b2 - ln2 * mlp_w2.sum(axis=1)).reshape(NI, 1, NF)
    l2b = lin2_b.reshape(NI, 1, HIDDEN)
    lb = (lin_b - ln2 * lin_w.sum(axis=1)).reshape(NI, 1, HIDDEN)
    pb = pool_b.reshape(1, EMB)

    split = _split

    embh, embl = split(emb_table)
    w1h, w1l = split(w1c)
    w2h, w2l = split(mlp_w2)
    l1h, l1l = split(lin1_w)
    l2h, l2l = split(lin2_w)
    lwh, lwl = split(lin_w)
    pwh, pwl = split(pool_w)

    def whole(a):
        return pl.BlockSpec(a.shape, lambda g: (0,) * a.ndim)

    hm = pl.pallas_call(
        _mol_kernel,
        grid=(G,),
        in_specs=[
            pl.BlockSpec((1, P, 1), lambda g: (g, 0, 0)),
            pl.BlockSpec((1, P, 3), lambda g: (g, 0, 0)),
            whole(embh), whole(embl),
            whole(w1h), whole(w1l), whole(b1c),
            whole(w2h), whole(w2l), whole(b2),
            whole(l1h), whole(l1l),
            whole(l2h), whole(l2l), whole(l2b),
            whole(lwh), whole(lwl), whole(lb),
        ],
        out_specs=pl.BlockSpec((1, 1, HIDDEN), lambda g: (g, 0, 0)),
        out_shape=jax.ShapeDtypeStruct((G, 1, HIDDEN), jnp.float32),
    )(z3, pos3, embh, embl, w1h, w1l, b1c, w2h, w2l, b2,
      l1h, l1l, l2h, l2l, l2b, lwh, lwl, lb)

    return pl.pallas_call(
        _pool_kernel,
        out_shape=jax.ShapeDtypeStruct((G, EMB), jnp.float32),
    )(hm.reshape(G, HIDDEN), pwh, pwl, pb)


# speedup vs baseline: 1.0772x; 1.0772x over previous
import jax
import jax.numpy as jnp
from jax.experimental import pallas as pl

G = 128
P = 64
HIDDEN = 128
NF = 128
NG = 50
NI = 6
CUTOFF = 10.0
EMB = 128

_BF = jnp.bfloat16
_F32 = jnp.float32


def _sp(x):
    return jnp.maximum(x, 0.0) + jnp.log1p(jnp.exp(-jnp.abs(x)))


def _split(a):
    bits = jax.lax.bitcast_convert_type(a, jnp.uint32)
    hi32 = jax.lax.bitcast_convert_type(bits & jnp.uint32(0xFFFF0000), _F32)
    return hi32.astype(_BF), (a - hi32).astype(_BF)


def _dot3(ah, al, bh, bl):
    return (jnp.dot(ah, bh, preferred_element_type=_F32)
            + jnp.dot(al, bh, preferred_element_type=_F32)
            + jnp.dot(ah, bl, preferred_element_type=_F32))


def _mol_kernel(z_ref, pos_ref, embh_ref, embl_ref, w1h_ref, w1l_ref, b1c_ref,
                w2h_ref, w2l_ref, b2_ref, l1h_ref, l1l_ref,
                l2h_ref, l2l_ref, l2b_ref, lwh_ref, lwl_ref, lb_ref, out_ref):
    p = pos_ref[0]
    zc = z_ref[0]

    tt = jax.lax.broadcasted_iota(jnp.int32, (P, 100), 1)
    oh = (zc == tt).astype(_BF)
    h = (jnp.dot(oh, embh_ref[...], preferred_element_type=_F32)
         + jnp.dot(oh, embl_ref[...], preferred_element_type=_F32))

    diff = p[:, None, :] - p[None, :, :]
    r2 = jnp.sum(diff * diff, axis=-1, keepdims=True)
    ewf = jnp.sqrt(r2 + 1e-12).reshape(P * P, 1)

    step = CUTOFF / (NG - 1)
    offs = jax.lax.broadcasted_iota(jnp.int32, (1, NG), 1).astype(_F32) * step
    gcoeff = -0.5 / (step * step)
    ea = jnp.exp(gcoeff * (ewf - offs) ** 2)

    cf = 0.5 * (jnp.cos(ewf * (jnp.pi / CUTOFF)) + 1.0)
    cf = cf * (ewf < CUTOFF).astype(_F32)
    ii = jax.lax.broadcasted_iota(jnp.int32, (P, P, 1), 0)
    jj = jax.lax.broadcasted_iota(jnp.int32, (P, P, 1), 1)
    scale = cf * (ii != jj).astype(_F32).reshape(P * P, 1)
    scale = jnp.broadcast_to(scale, (P * P, NF))

    eah, eal = _split(ea)
    h1c = _sp(_dot3(eah, eal, w1h_ref[...], w1l_ref[...])
              + b1c_ref[...])
    h1h, h1l = _split(h1c)

    for t in range(NI):
        sl = slice(t * NF, (t + 1) * NF)
        w = _dot3(h1h[:, sl], h1l[:, sl], w2h_ref[t], w2l_ref[t]) + b2_ref[t]
        w = w * scale
        x = _dot3(*_split(h), l1h_ref[t], l1l_ref[t])
        w3 = w.reshape(P, P, NF)
        agg = jnp.sum(w3 * x[None, :, :], axis=1)
        x2 = _sp(_dot3(*_split(agg), l2h_ref[t], l2l_ref[t]) + l2b_ref[t])
        x2 = _dot3(*_split(x2), lwh_ref[t], lwl_ref[t]) + lb_ref[t]
        h = h + x2

    out_ref[0] = jnp.sum(h, axis=0, keepdims=True) * (1.0 / P)


def _pool_kernel(hm_ref, pwh_ref, pwl_ref, pb_ref, out_ref):
    out_ref[...] = _dot3(*_split(hm_ref[...]), pwh_ref[...], pwl_ref[...]) \
        + pb_ref[...]


def kernel(z, pos, batch, emb_table, mlp_w1, mlp_b1, mlp_w2, mlp_b2,
           lin1_w, lin2_w, lin2_b, lin_w, lin_b, pool_w, pool_b):
    del batch
    ln2 = 0.6931471805599453
    z3 = z.reshape(G, P, 1).astype(jnp.int32)
    pos3 = pos.reshape(G, P, 3)
    w1c = jnp.transpose(mlp_w1, (1, 0, 2)).reshape(NG, NI * NF)
    b1c = mlp_b1.reshape(1, NI * NF)
    b2 = (mlp_b2 - ln2 * mlp_w2.sum(axis=1)).reshape(NI, 1, NF)
    l2b = lin2_b.reshape(NI, 1, HIDDEN)
    lb = (lin_b - ln2 * lin_w.sum(axis=1)).reshape(NI, 1, HIDDEN)
    pb = pool_b.reshape(1, EMB)

    split = _split

    embh, embl = split(emb_table)
    w1h, w1l = split(w1c)
    w2h, w2l = split(mlp_w2)
    l1h, l1l = split(lin1_w)
    l2h, l2l = split(lin2_w)
    lwh, lwl = split(lin_w)
    pwh, pwl = split(pool_w)

    def whole(a):
        return pl.BlockSpec(a.shape, lambda g: (0,) * a.ndim)

    hm = pl.pallas_call(
        _mol_kernel,
        grid=(G,),
        in_specs=[
            pl.BlockSpec((1, P, 1), lambda g: (g, 0, 0)),
            pl.BlockSpec((1, P, 3), lambda g: (g, 0, 0)),
            whole(embh), whole(embl),
            whole(w1h), whole(w1l), whole(b1c),
            whole(w2h), whole(w2l), whole(b2),
            whole(l1h), whole(l1l),
            whole(l2h), whole(l2l), whole(l2b),
            whole(lwh), whole(lwl), whole(lb),
        ],
        out_specs=pl.BlockSpec((1, 1, HIDDEN), lambda g: (g, 0, 0)),
        out_shape=jax.ShapeDtypeStruct((G, 1, HIDDEN), jnp.float32),
    )(z3, pos3, embh, embl, w1h, w1l, b1c, w2h, w2l, b2,
      l1h, l1l, l2h, l2l, l2b, lwh, lwl, lb)

    return pl.pallas_call(
        _pool_kernel,
        out_shape=jax.ShapeDtypeStruct((G, EMB), jnp.float32),
    )(hm.reshape(G, HIDDEN), pwh, pwl, pb)
